# Initial kernel scaffold; baseline (speedup 1.0000x reference)
import functools

import jax
import jax.numpy as jnp
from jax.experimental import pallas as pl
from jax.experimental.pallas import tpu as pltpu

_N = 10000
_E = 160000
_W = 64
_DEPTH = 3

_KF_BLK = 256
_MV_BLK = 640
_ND_BLK = 1000


def _kform_body(ea, k1w, k1b, k2w, k2b, k3w, k3b, out):
    t = jnp.maximum(jnp.dot(ea[...], k1w[...], preferred_element_type=jnp.float32) + k1b[...], 0.0)
    t = jnp.maximum(jnp.dot(t, k2w[...], preferred_element_type=jnp.float32) + k2b[...], 0.0)
    out[...] = jnp.dot(t, k3w[...], preferred_element_type=jnp.float32) + k3b[...]


def _kform(ea, k1w, k1b, k2w, k2b, k3w, k3b):
    n_blk = _E // _KF_BLK
    full = lambda *shape: pl.BlockSpec(shape, lambda i, s=len(shape): (0,) * s)
    return pl.pallas_call(
        _kform_body,
        grid=(n_blk,),
        in_specs=[
            pl.BlockSpec((_KF_BLK, 16), lambda i: (i, 0)),
            full(16, 256), full(1, 256),
            full(256, 512), full(1, 512),
            full(512, 4096), full(1, 4096),
        ],
        out_specs=pl.BlockSpec((_KF_BLK, 4096), lambda i: (i, 0)),
        out_shape=jax.ShapeDtypeStruct((_E, 4096), jnp.float32),
    )(ea, k1w, k1b, k2w, k2b, k3w, k3b)


def _matvec_body(xj, K, out):
    x = xj[...]
    acc = x[:, 0:1] * K[:, 0:_W]
    for f in range(1, _W):
        acc = acc + x[:, f:f + 1] * K[:, _W * f:_W * (f + 1)]
    out[...] = acc


def _matvec(xj, K):
    n_blk = _E // _MV_BLK
    return pl.pallas_call(
        _matvec_body,
        grid=(n_blk,),
        in_specs=[
            pl.BlockSpec((_MV_BLK, _W), lambda i: (i, 0)),
            pl.BlockSpec((_MV_BLK, 4096), lambda i: (i, 0)),
        ],
        out_specs=pl.BlockSpec((_MV_BLK, _W), lambda i: (i, 0)),
        out_shape=jax.ShapeDtypeStruct((_E, _W), jnp.float32),
    )(xj, K)


def _update_body(agg, h, deg, rootw, rootb, out):
    hr = jnp.dot(h[...], rootw[...], preferred_element_type=jnp.float32) + rootb[...]
    out[...] = jnp.maximum(agg[...] / deg[...] + hr, 0.0)


def _update(agg, h, deg, rootw, rootb):
    n_blk = _N // _ND_BLK
    return pl.pallas_call(
        _update_body,
        grid=(n_blk,),
        in_specs=[
            pl.BlockSpec((_ND_BLK, _W), lambda i: (i, 0)),
            pl.BlockSpec((_ND_BLK, _W), lambda i: (i, 0)),
            pl.BlockSpec((_ND_BLK, 1), lambda i: (i, 0)),
            pl.BlockSpec((_W, _W), lambda i: (0, 0)),
            pl.BlockSpec((1, _W), lambda i: (0, 0)),
        ],
        out_specs=pl.BlockSpec((_ND_BLK, _W), lambda i: (i, 0)),
        out_shape=jax.ShapeDtypeStruct((_N, _W), jnp.float32),
    )(agg, h, deg, rootw, rootb)


def _h0_body(x, w, b, out):
    out[...] = x[...] * w[...] + b[...]


def _h0(x, w, b):
    n_blk = _N // _ND_BLK
    return pl.pallas_call(
        _h0_body,
        grid=(n_blk,),
        in_specs=[
            pl.BlockSpec((_ND_BLK, 1), lambda i: (i, 0)),
            pl.BlockSpec((1, _W), lambda i: (0, 0)),
            pl.BlockSpec((1, _W), lambda i: (0, 0)),
        ],
        out_specs=pl.BlockSpec((_ND_BLK, _W), lambda i: (i, 0)),
        out_shape=jax.ShapeDtypeStruct((_N, _W), jnp.float32),
    )(x, w, b)


def _fc2_body(h, w, b, out):
    out[...] = jnp.sum(h[...] * w[...], axis=1, keepdims=True) + b[...]


def _fc2(h, w, b):
    n_blk = _N // _ND_BLK
    return pl.pallas_call(
        _fc2_body,
        grid=(n_blk,),
        in_specs=[
            pl.BlockSpec((_ND_BLK, _W), lambda i: (i, 0)),
            pl.BlockSpec((1, _W), lambda i: (0, 0)),
            pl.BlockSpec((1, 1), lambda i: (0, 0)),
        ],
        out_specs=pl.BlockSpec((_ND_BLK, 1), lambda i: (i, 0)),
        out_shape=jax.ShapeDtypeStruct((_N, 1), jnp.float32),
    )(h, w, b)


def kernel(x, edge_index, edge_attr, fc1_W, fc1_b, k1_W, k1_b, k2_W, k2_b,
           k3_W, k3_b, root_W, root_b, fc2_W, fc2_b):
    src = edge_index[0]
    dst = edge_index[1]

    K = _kform(edge_attr, k1_W, k1_b.reshape(1, -1), k2_W, k2_b.reshape(1, -1),
               k3_W, k3_b.reshape(1, -1))

    h = _h0(x, fc1_W, fc1_b.reshape(1, -1))

    deg = jnp.maximum(
        jnp.zeros((_N,), jnp.float32).at[dst].add(1.0), 1.0).reshape(_N, 1)

    for _ in range(_DEPTH):
        xj = jnp.take(h, src, axis=0)
        msg = _matvec(xj, K)
        agg = jnp.zeros((_N, _W), jnp.float32).at[dst].add(msg)
        h = _update(agg, h, deg, root_W, root_b.reshape(1, -1))

    return _fc2(h, fc2_W, fc2_b.reshape(1, -1))


# speedup vs baseline: 1.5454x; 1.5454x over previous
import functools

import jax
import jax.numpy as jnp
from jax.experimental import pallas as pl
from jax.experimental.pallas import tpu as pltpu

_N = 10000
_E = 160000
_W = 64
_DEPTH = 3

_KF_BLK = 256
_MV_BLK = 640
_ND_BLK = 1000


def _kform_body(ea, k1w, k1b, k2w, k2b, k3w, k3b, out):
    t = jnp.maximum(jnp.dot(ea[...], k1w[...], preferred_element_type=jnp.float32) + k1b[...], 0.0)
    t = jnp.maximum(jnp.dot(t, k2w[...], preferred_element_type=jnp.float32) + k2b[...], 0.0)
    out[...] = jnp.dot(t, k3w[...], preferred_element_type=jnp.float32) + k3b[...]


def _kform(ea, k1w, k1b, k2w, k2b, k3w, k3b):
    n_blk = _E // _KF_BLK
    full = lambda *shape: pl.BlockSpec(shape, lambda i, s=len(shape): (0,) * s)
    return pl.pallas_call(
        _kform_body,
        grid=(n_blk,),
        in_specs=[
            pl.BlockSpec((_KF_BLK, 16), lambda i: (i, 0)),
            full(16, 256), full(1, 256),
            full(256, 512), full(1, 512),
            full(512, 4096), full(1, 4096),
        ],
        out_specs=pl.BlockSpec((_KF_BLK, 4096), lambda i: (i, 0)),
        out_shape=jax.ShapeDtypeStruct((_E, 4096), jnp.float32),
    )(ea, k1w, k1b, k2w, k2b, k3w, k3b)


def _matvec_body(xj, K, out):
    x = xj[...]
    acc = x[:, 0:1] * K[:, 0:_W]
    for f in range(1, _W):
        acc = acc + x[:, f:f + 1] * K[:, _W * f:_W * (f + 1)]
    out[...] = acc


def _matvec(xj, K):
    n_blk = _E // _MV_BLK
    return pl.pallas_call(
        _matvec_body,
        grid=(n_blk,),
        in_specs=[
            pl.BlockSpec((_MV_BLK, _W), lambda i: (i, 0)),
            pl.BlockSpec((_MV_BLK, 4096), lambda i: (i, 0)),
        ],
        out_specs=pl.BlockSpec((_MV_BLK, _W), lambda i: (i, 0)),
        out_shape=jax.ShapeDtypeStruct((_E, _W), jnp.float32),
    )(xj, K)


def _update_body(agg, h, deg, rootw, rootb, out):
    hr = jnp.dot(h[...], rootw[...], preferred_element_type=jnp.float32) + rootb[...]
    out[...] = jnp.maximum(agg[...] / deg[...] + hr, 0.0)


def _update(agg, h, deg, rootw, rootb):
    n_blk = _N // _ND_BLK
    return pl.pallas_call(
        _update_body,
        grid=(n_blk,),
        in_specs=[
            pl.BlockSpec((_ND_BLK, _W), lambda i: (i, 0)),
            pl.BlockSpec((_ND_BLK, _W), lambda i: (i, 0)),
            pl.BlockSpec((_ND_BLK, 1), lambda i: (i, 0)),
            pl.BlockSpec((_W, _W), lambda i: (0, 0)),
            pl.BlockSpec((1, _W), lambda i: (0, 0)),
        ],
        out_specs=pl.BlockSpec((_ND_BLK, _W), lambda i: (i, 0)),
        out_shape=jax.ShapeDtypeStruct((_N, _W), jnp.float32),
    )(agg, h, deg, rootw, rootb)


def _h0_body(x, w, b, out):
    out[...] = x[...] * w[...] + b[...]


def _h0(x, w, b):
    n_blk = _N // _ND_BLK
    return pl.pallas_call(
        _h0_body,
        grid=(n_blk,),
        in_specs=[
            pl.BlockSpec((_ND_BLK, 1), lambda i: (i, 0)),
            pl.BlockSpec((1, _W), lambda i: (0, 0)),
            pl.BlockSpec((1, _W), lambda i: (0, 0)),
        ],
        out_specs=pl.BlockSpec((_ND_BLK, _W), lambda i: (i, 0)),
        out_shape=jax.ShapeDtypeStruct((_N, _W), jnp.float32),
    )(x, w, b)


def _fc2_body(h, w, b, out):
    out[...] = jnp.sum(h[...] * w[...], axis=1, keepdims=True) + b[...]


def _fc2(h, w, b):
    n_blk = _N // _ND_BLK
    return pl.pallas_call(
        _fc2_body,
        grid=(n_blk,),
        in_specs=[
            pl.BlockSpec((_ND_BLK, _W), lambda i: (i, 0)),
            pl.BlockSpec((1, _W), lambda i: (0, 0)),
            pl.BlockSpec((1, 1), lambda i: (0, 0)),
        ],
        out_specs=pl.BlockSpec((_ND_BLK, 1), lambda i: (i, 0)),
        out_shape=jax.ShapeDtypeStruct((_N, 1), jnp.float32),
    )(h, w, b)


def kernel(x, edge_index, edge_attr, fc1_W, fc1_b, k1_W, k1_b, k2_W, k2_b,
           k3_W, k3_b, root_W, root_b, fc2_W, fc2_b):
    src = edge_index[0]
    dst = edge_index[1]

    K = _kform(edge_attr, k1_W, k1_b.reshape(1, -1), k2_W, k2_b.reshape(1, -1),
               k3_W, k3_b.reshape(1, -1))

    h = _h0(x, fc1_W, fc1_b.reshape(1, -1))

    deg = jnp.maximum(
        jnp.zeros((_N,), jnp.float32).at[dst].add(1.0), 1.0).reshape(_N, 1)

    for _ in range(_DEPTH):
        xj = jnp.take(h, src, axis=0)
        msg = _matvec(xj, K)
        agg = jnp.zeros((_N, _W), jnp.float32).at[dst].add(msg)
        h = _update(agg, h, deg, root_W, root_b.reshape(1, -1))

    return _fc2(h, fc2_W.reshape(1, _W), fc2_b.reshape(1, 1))


# speedup vs baseline: 1.8053x; 1.1682x over previous
import functools

import jax
import jax.numpy as jnp
from jax import lax
from jax.experimental import pallas as pl
from jax.experimental.pallas import tpu as pltpu
from jax.experimental.pallas import tpu_sc as plsc

_N = 10000
_E = 160000
_W = 64
_DEPTH = 3

_NC = 2
_NS = 16
_NW = _NC * _NS
_EPW = _E // _NW
_GCH = 128
_NFULL = _EPW // _GCH
_TAIL = _EPW - _NFULL * _GCH
_RPT = _N // _NS

_KF_BLK = 256
_MV_BLK = 640
_ND_BLK = 1000


def _kform_body(ea, k1w, k1b, k2w, k2b, k3w, k3b, out):
    t = jnp.maximum(jnp.dot(ea[...], k1w[...], preferred_element_type=jnp.float32) + k1b[...], 0.0)
    t = jnp.maximum(jnp.dot(t, k2w[...], preferred_element_type=jnp.float32) + k2b[...], 0.0)
    out[...] = jnp.dot(t, k3w[...], preferred_element_type=jnp.float32) + k3b[...]


def _kform(ea, k1w, k1b, k2w, k2b, k3w, k3b):
    n_blk = _E // _KF_BLK
    full = lambda *shape: pl.BlockSpec(shape, lambda i, s=len(shape): (0,) * s)
    return pl.pallas_call(
        _kform_body,
        grid=(n_blk,),
        in_specs=[
            pl.BlockSpec((_KF_BLK, 16), lambda i: (i, 0)),
            full(16, 256), full(1, 256),
            full(256, 512), full(1, 512),
            full(512, 4096), full(1, 4096),
        ],
        out_specs=pl.BlockSpec((_KF_BLK, 4096), lambda i: (i, 0)),
        out_shape=jax.ShapeDtypeStruct((_E, 4096), jnp.float32),
    )(ea, k1w, k1b, k2w, k2b, k3w, k3b)


def _matvec_body(xj, K, out):
    x = xj[...]
    acc = x[:, 0:1] * K[:, 0:_W]
    for f in range(1, _W):
        acc = acc + x[:, f:f + 1] * K[:, _W * f:_W * (f + 1)]
    out[...] = acc


def _matvec(xj, K):
    n_blk = _E // _MV_BLK
    return pl.pallas_call(
        _matvec_body,
        grid=(n_blk,),
        in_specs=[
            pl.BlockSpec((_MV_BLK, _W), lambda i: (i, 0)),
            pl.BlockSpec((_MV_BLK, 4096), lambda i: (i, 0)),
        ],
        out_specs=pl.BlockSpec((_MV_BLK, _W), lambda i: (i, 0)),
        out_shape=jax.ShapeDtypeStruct((_E, _W), jnp.float32),
    )(xj, K)


def _update_body(a0, a1, d0, d1, h, rootw, rootb, out):
    hr = jnp.dot(h[...], rootw[...], preferred_element_type=jnp.float32) + rootb[...]
    deg = jnp.maximum(d0[...] + d1[...], 1.0)
    out[...] = jnp.maximum((a0[...] + a1[...]) / deg + hr, 0.0)


def _update(a0, a1, d0, d1, h, rootw, rootb):
    n_blk = _N // _ND_BLK
    nd = pl.BlockSpec((_ND_BLK, _W), lambda i: (i, 0))
    return pl.pallas_call(
        _update_body,
        grid=(n_blk,),
        in_specs=[
            nd, nd, nd, nd, nd,
            pl.BlockSpec((_W, _W), lambda i: (0, 0)),
            pl.BlockSpec((1, _W), lambda i: (0, 0)),
        ],
        out_specs=pl.BlockSpec((_ND_BLK, _W), lambda i: (i, 0)),
        out_shape=jax.ShapeDtypeStruct((_N, _W), jnp.float32),
    )(a0, a1, d0, d1, h, rootw, rootb)


def _h0_body(x, w, b, out):
    out[...] = x[...] * w[...] + b[...]


def _h0(x, w, b):
    n_blk = _N // _ND_BLK
    return pl.pallas_call(
        _h0_body,
        grid=(n_blk,),
        in_specs=[
            pl.BlockSpec((_ND_BLK, 1), lambda i: (i, 0)),
            pl.BlockSpec((1, _W), lambda i: (0, 0)),
            pl.BlockSpec((1, _W), lambda i: (0, 0)),
        ],
        out_specs=pl.BlockSpec((_ND_BLK, _W), lambda i: (i, 0)),
        out_shape=jax.ShapeDtypeStruct((_N, _W), jnp.float32),
    )(x, w, b)


def _fc2_body(h, w, b, out):
    out[...] = jnp.sum(h[...] * w[...], axis=1, keepdims=True) + b[...]


def _fc2(h, w, b):
    n_blk = _N // _ND_BLK
    return pl.pallas_call(
        _fc2_body,
        grid=(n_blk,),
        in_specs=[
            pl.BlockSpec((_ND_BLK, _W), lambda i: (i, 0)),
            pl.BlockSpec((1, _W), lambda i: (0, 0)),
            pl.BlockSpec((1, 1), lambda i: (0, 0)),
        ],
        out_specs=pl.BlockSpec((_ND_BLK, 1), lambda i: (i, 0)),
        out_shape=jax.ShapeDtypeStruct((_N, 1), jnp.float32),
    )(h, w, b)


def _gather_body(h_hbm, src_hbm, out_hbm, idx_f, idx_t, rows_v, sem):
    wid = lax.axis_index("s") * _NC + lax.axis_index("c")
    base = wid * _EPW

    def chunk(j, _):
        off = base + j * _GCH
        pltpu.sync_copy(src_hbm.at[pl.ds(off, _GCH)], idx_f)
        pltpu.async_copy(h_hbm.at[idx_f], rows_v, sem).wait()
        pltpu.sync_copy(rows_v, out_hbm.at[pl.ds(off, _GCH)])
        return 0

    lax.fori_loop(0, _NFULL, chunk, 0)
    off = base + _NFULL * _GCH
    pltpu.sync_copy(src_hbm.at[pl.ds(off, _TAIL)], idx_t)
    pltpu.async_copy(h_hbm.at[idx_t], rows_v.at[pl.ds(0, _TAIL)], sem).wait()
    pltpu.sync_copy(rows_v.at[pl.ds(0, _TAIL)], out_hbm.at[pl.ds(off, _TAIL)])


def _scatter_body(msg_hbm, dst_hbm, zeros_hbm, out_hbm, idx_f, idx_t, msg_v,
                  agg_sh):
    c = lax.axis_index("c")
    s = lax.axis_index("s")
    wid = s * _NC + c
    pltpu.sync_copy(zeros_hbm.at[pl.ds(s * _RPT, _RPT)],
                    agg_sh.at[pl.ds(s * _RPT, _RPT)])
    plsc.subcore_barrier()
    base = wid * _EPW

    def chunk(j, _):
        off = base + j * _GCH
        pltpu.sync_copy(dst_hbm.at[pl.ds(off, _GCH)], idx_f)
        pltpu.sync_copy(msg_hbm.at[pl.ds(off, _GCH)], msg_v)
        pltpu.sync_copy(msg_v, agg_sh.at[idx_f], add=True)
        return 0

    lax.fori_loop(0, _NFULL, chunk, 0)
    off = base + _NFULL * _GCH
    pltpu.sync_copy(dst_hbm.at[pl.ds(off, _TAIL)], idx_t)
    pltpu.sync_copy(msg_hbm.at[pl.ds(off, _TAIL)], msg_v.at[pl.ds(0, _TAIL)])
    pltpu.sync_copy(msg_v.at[pl.ds(0, _TAIL)], agg_sh.at[idx_t], add=True)
    plsc.subcore_barrier()
    pltpu.sync_copy(agg_sh.at[pl.ds(s * _RPT, _RPT)],
                    out_hbm.at[c, pl.ds(s * _RPT, _RPT)])


@functools.cache
def _sc_kernels():
    mesh = plsc.VectorSubcoreMesh(core_axis_name="c", subcore_axis_name="s",
                                  num_cores=_NC, num_subcores=_NS)
    gather = pl.kernel(
        _gather_body,
        out_type=jax.ShapeDtypeStruct((_E, _W), jnp.float32),
        mesh=mesh,
        compiler_params=pltpu.CompilerParams(use_tc_tiling_on_sc=False),
        scratch_types=[
            pltpu.VMEM((_GCH,), jnp.int32),
            pltpu.VMEM((_TAIL,), jnp.int32),
            pltpu.VMEM((_GCH, _W), jnp.float32),
            pltpu.SemaphoreType.DMA,
        ],
    )
    scatter = pl.kernel(
        _scatter_body,
        out_type=jax.ShapeDtypeStruct((_NC, _N, _W), jnp.float32),
        mesh=mesh,
        compiler_params=pltpu.CompilerParams(use_tc_tiling_on_sc=False),
        scratch_types=[
            pltpu.VMEM((_GCH,), jnp.int32),
            pltpu.VMEM((_TAIL,), jnp.int32),
            pltpu.VMEM((_GCH, _W), jnp.float32),
            pltpu.VMEM_SHARED((_N, _W), jnp.float32),
        ],
    )
    return gather, scatter


def kernel(x, edge_index, edge_attr, fc1_W, fc1_b, k1_W, k1_b, k2_W, k2_b,
           k3_W, k3_b, root_W, root_b, fc2_W, fc2_b):
    src = edge_index[0]
    dst = edge_index[1]

    K = _kform(edge_attr, k1_W, k1_b.reshape(1, -1), k2_W, k2_b.reshape(1, -1),
               k3_W, k3_b.reshape(1, -1))

    h = _h0(x, fc1_W, fc1_b.reshape(1, -1))

    _sc_gather, _sc_scatter = _sc_kernels()
    zeros_nw = jnp.zeros((_N, _W), jnp.float32)
    ones_ew = jnp.ones((_E, _W), jnp.float32)
    degp = _sc_scatter(ones_ew, dst, zeros_nw)
    d0, d1 = degp[0], degp[1]

    for _ in range(_DEPTH):
        xj = _sc_gather(h, src)
        msg = _matvec(xj, K)
        aggp = _sc_scatter(msg, dst, zeros_nw)
        h = _update(aggp[0], aggp[1], d0, d1, h, root_W, root_b.reshape(1, -1))

    return _fc2(h, fc2_W.reshape(1, _W), fc2_b.reshape(1, 1))


# speedup vs baseline: 2.4375x; 1.3502x over previous
import functools

import jax
import jax.numpy as jnp
import numpy as np
from jax import lax
from jax.experimental import pallas as pl
from jax.experimental.pallas import tpu as pltpu
from jax.experimental.pallas import tpu_sc as plsc

_N = 10000
_E = 160000
_W = 64
_DEPTH = 3

_NC = 2
_NS = 16
_NW = _NC * _NS
_EPW = _E // _NW
_GCH = 128
_NFULL = _EPW // _GCH
_TAIL = _EPW - _NFULL * _GCH
_RPT = _N // _NS

_PERM = np.empty((4096,), np.int32)
for _i in range(32):
    for _p in range(2):
        for _g in range(64):
            _PERM[128 * _i + 64 * _p + _g] = (2 * _i + _p) * 64 + _g

_KF_BLK = 256
_MV_BLK = 640
_ND_BLK = 1000


def _kform_body(ea, k1w, k1b, k2w, k2b, k3w, k3b, out):
    t = jnp.maximum(jnp.dot(ea[...], k1w[...], preferred_element_type=jnp.float32) + k1b[...], 0.0)
    t = jnp.maximum(jnp.dot(t, k2w[...], preferred_element_type=jnp.float32) + k2b[...], 0.0)
    out[...] = (jnp.dot(t, k3w[...], preferred_element_type=jnp.float32)
                + k3b[...]).astype(jnp.bfloat16)


def _kform(ea, k1w, k1b, k2w, k2b, k3w, k3b):
    n_blk = _E // _KF_BLK
    full = lambda *shape: pl.BlockSpec(shape, lambda i, s=len(shape): (0,) * s)
    return pl.pallas_call(
        _kform_body,
        grid=(n_blk,),
        in_specs=[
            pl.BlockSpec((_KF_BLK, 16), lambda i: (i, 0)),
            full(16, 256), full(1, 256),
            full(256, 512), full(1, 512),
            full(512, 4096), full(1, 4096),
        ],
        out_specs=pl.BlockSpec((_KF_BLK, 4096), lambda i: (i, 0)),
        out_shape=jax.ShapeDtypeStruct((_E, 4096), jnp.bfloat16),
    )(ea, k1w, k1b, k2w, k2b, k3w, k3b)


def _matvec_body(xj, K, out):
    x = xj[...]
    acc = jnp.zeros((_MV_BLK, 2 * _W), jnp.float32)
    for i in range(_W // 2):
        kk = K[:, 128 * i:128 * (i + 1)].astype(jnp.float32)
        xa = jax.lax.broadcast_in_dim(x[:, 2 * i:2 * i + 1], (_MV_BLK, _W), (0, 1))
        xb = jax.lax.broadcast_in_dim(x[:, 2 * i + 1:2 * i + 2], (_MV_BLK, _W), (0, 1))
        xi = jnp.concatenate([xa, xb], axis=1)
        acc = acc + xi * kk
    out[...] = acc[:, :_W] + acc[:, _W:]


def _matvec(xj, K):
    n_blk = _E // _MV_BLK
    return pl.pallas_call(
        _matvec_body,
        grid=(n_blk,),
        in_specs=[
            pl.BlockSpec((_MV_BLK, _W), lambda i: (i, 0)),
            pl.BlockSpec((_MV_BLK, 4096), lambda i: (i, 0)),
        ],
        out_specs=pl.BlockSpec((_MV_BLK, _W), lambda i: (i, 0)),
        out_shape=jax.ShapeDtypeStruct((_E, _W), jnp.float32),
    )(xj, K)


def _update_body(a0, a1, d0, d1, h, rootw, rootb, out):
    hr = jnp.dot(h[...], rootw[...], preferred_element_type=jnp.float32) + rootb[...]
    deg = jnp.maximum(d0[...] + d1[...], 1.0)
    out[...] = jnp.maximum((a0[...] + a1[...]) / deg + hr, 0.0)


def _update(a0, a1, d0, d1, h, rootw, rootb):
    n_blk = _N // _ND_BLK
    nd = pl.BlockSpec((_ND_BLK, _W), lambda i: (i, 0))
    return pl.pallas_call(
        _update_body,
        grid=(n_blk,),
        in_specs=[
            nd, nd, nd, nd, nd,
            pl.BlockSpec((_W, _W), lambda i: (0, 0)),
            pl.BlockSpec((1, _W), lambda i: (0, 0)),
        ],
        out_specs=pl.BlockSpec((_ND_BLK, _W), lambda i: (i, 0)),
        out_shape=jax.ShapeDtypeStruct((_N, _W), jnp.float32),
    )(a0, a1, d0, d1, h, rootw, rootb)


def _h0_body(x, w, b, out):
    out[...] = x[...] * w[...] + b[...]


def _h0(x, w, b):
    n_blk = _N // _ND_BLK
    return pl.pallas_call(
        _h0_body,
        grid=(n_blk,),
        in_specs=[
            pl.BlockSpec((_ND_BLK, 1), lambda i: (i, 0)),
            pl.BlockSpec((1, _W), lambda i: (0, 0)),
            pl.BlockSpec((1, _W), lambda i: (0, 0)),
        ],
        out_specs=pl.BlockSpec((_ND_BLK, _W), lambda i: (i, 0)),
        out_shape=jax.ShapeDtypeStruct((_N, _W), jnp.float32),
    )(x, w, b)


def _fc2_body(h, w, b, out):
    out[...] = jnp.sum(h[...] * w[...], axis=1, keepdims=True) + b[...]


def _fc2(h, w, b):
    n_blk = _N // _ND_BLK
    return pl.pallas_call(
        _fc2_body,
        grid=(n_blk,),
        in_specs=[
            pl.BlockSpec((_ND_BLK, _W), lambda i: (i, 0)),
            pl.BlockSpec((1, _W), lambda i: (0, 0)),
            pl.BlockSpec((1, 1), lambda i: (0, 0)),
        ],
        out_specs=pl.BlockSpec((_ND_BLK, 1), lambda i: (i, 0)),
        out_shape=jax.ShapeDtypeStruct((_N, 1), jnp.float32),
    )(h, w, b)


def _gather_body(h_hbm, src_hbm, out_hbm, idx_f, idx_t, rows_v, sem):
    wid = lax.axis_index("s") * _NC + lax.axis_index("c")
    base = wid * _EPW

    def chunk(j, _):
        off = base + j * _GCH
        pltpu.sync_copy(src_hbm.at[pl.ds(off, _GCH)], idx_f)
        pltpu.async_copy(h_hbm.at[idx_f], rows_v, sem).wait()
        pltpu.sync_copy(rows_v, out_hbm.at[pl.ds(off, _GCH)])
        return 0

    lax.fori_loop(0, _NFULL, chunk, 0)
    off = base + _NFULL * _GCH
    pltpu.sync_copy(src_hbm.at[pl.ds(off, _TAIL)], idx_t)
    pltpu.async_copy(h_hbm.at[idx_t], rows_v.at[pl.ds(0, _TAIL)], sem).wait()
    pltpu.sync_copy(rows_v.at[pl.ds(0, _TAIL)], out_hbm.at[pl.ds(off, _TAIL)])


def _scatter_body(msg_hbm, dst_hbm, zeros_hbm, out_hbm, idx_f, idx_t, msg_v,
                  agg_sh):
    c = lax.axis_index("c")
    s = lax.axis_index("s")
    wid = s * _NC + c
    pltpu.sync_copy(zeros_hbm.at[pl.ds(s * _RPT, _RPT)],
                    agg_sh.at[pl.ds(s * _RPT, _RPT)])
    plsc.subcore_barrier()
    base = wid * _EPW

    def chunk(j, _):
        off = base + j * _GCH
        pltpu.sync_copy(dst_hbm.at[pl.ds(off, _GCH)], idx_f)
        pltpu.sync_copy(msg_hbm.at[pl.ds(off, _GCH)], msg_v)
        pltpu.sync_copy(msg_v, agg_sh.at[idx_f], add=True)
        return 0

    lax.fori_loop(0, _NFULL, chunk, 0)
    off = base + _NFULL * _GCH
    pltpu.sync_copy(dst_hbm.at[pl.ds(off, _TAIL)], idx_t)
    pltpu.sync_copy(msg_hbm.at[pl.ds(off, _TAIL)], msg_v.at[pl.ds(0, _TAIL)])
    pltpu.sync_copy(msg_v.at[pl.ds(0, _TAIL)], agg_sh.at[idx_t], add=True)
    plsc.subcore_barrier()
    pltpu.sync_copy(agg_sh.at[pl.ds(s * _RPT, _RPT)],
                    out_hbm.at[c, pl.ds(s * _RPT, _RPT)])


@functools.cache
def _sc_kernels():
    mesh = plsc.VectorSubcoreMesh(core_axis_name="c", subcore_axis_name="s",
                                  num_cores=_NC, num_subcores=_NS)
    gather = pl.kernel(
        _gather_body,
        out_type=jax.ShapeDtypeStruct((_E, _W), jnp.float32),
        mesh=mesh,
        compiler_params=pltpu.CompilerParams(use_tc_tiling_on_sc=False),
        scratch_types=[
            pltpu.VMEM((_GCH,), jnp.int32),
            pltpu.VMEM((_TAIL,), jnp.int32),
            pltpu.VMEM((_GCH, _W), jnp.float32),
            pltpu.SemaphoreType.DMA,
        ],
    )
    scatter = pl.kernel(
        _scatter_body,
        out_type=jax.ShapeDtypeStruct((_NC, _N, _W), jnp.float32),
        mesh=mesh,
        compiler_params=pltpu.CompilerParams(use_tc_tiling_on_sc=False),
        scratch_types=[
            pltpu.VMEM((_GCH,), jnp.int32),
            pltpu.VMEM((_TAIL,), jnp.int32),
            pltpu.VMEM((_GCH, _W), jnp.float32),
            pltpu.VMEM_SHARED((_N, _W), jnp.float32),
        ],
    )
    return gather, scatter


def kernel(x, edge_index, edge_attr, fc1_W, fc1_b, k1_W, k1_b, k2_W, k2_b,
           k3_W, k3_b, root_W, root_b, fc2_W, fc2_b):
    src = edge_index[0]
    dst = edge_index[1]

    K = _kform(edge_attr, k1_W, k1_b.reshape(1, -1), k2_W, k2_b.reshape(1, -1),
               k3_W[:, _PERM], k3_b[_PERM].reshape(1, -1))

    h = _h0(x, fc1_W, fc1_b.reshape(1, -1))

    _sc_gather, _sc_scatter = _sc_kernels()
    zeros_nw = jnp.zeros((_N, _W), jnp.float32)
    ones_ew = jnp.ones((_E, _W), jnp.float32)
    degp = _sc_scatter(ones_ew, dst, zeros_nw)
    d0, d1 = degp[0], degp[1]

    for _ in range(_DEPTH):
        xj = _sc_gather(h, src)
        msg = _matvec(xj, K)
        aggp = _sc_scatter(msg, dst, zeros_nw)
        h = _update(aggp[0], aggp[1], d0, d1, h, root_W, root_b.reshape(1, -1))

    return _fc2(h, fc2_W.reshape(1, _W), fc2_b.reshape(1, 1))


# speedup vs baseline: 4.6653x; 1.9140x over previous
import functools

import jax
import jax.numpy as jnp
import numpy as np
from jax import lax
from jax.experimental import pallas as pl
from jax.experimental.pallas import tpu as pltpu
from jax.experimental.pallas import tpu_sc as plsc

_N = 10000
_E = 160000
_W = 64
_DEPTH = 3

_NC = 2
_NS = 16
_NW = _NC * _NS
_EPW = _E // _NW
_GCH = 128
_NFULL = _EPW // _GCH
_TAIL = _EPW - _NFULL * _GCH
_RPT = _N // _NS

_PERM = np.empty((4096,), np.int32)
for _i in range(32):
    for _p in range(2):
        for _g in range(64):
            _PERM[128 * _i + 64 * _p + _g] = (2 * _i + _p) * 64 + _g

_BX = np.zeros((64, 4096), np.float32)
for _i in range(32):
    for _p in range(2):
        _BX[2 * _i + _p, 128 * _i + 64 * _p:128 * _i + 64 * _p + 64] = 1.0

_EB = 128

_KF_BLK = 256
_MV_BLK = 640
_ND_BLK = 1000


def _kform_body(ea, k1w, k1b, k2w, k2b, k3w, k3b, out):
    t = jnp.maximum(jnp.dot(ea[...], k1w[...], preferred_element_type=jnp.float32) + k1b[...], 0.0)
    t = jnp.maximum(jnp.dot(t, k2w[...], preferred_element_type=jnp.float32) + k2b[...], 0.0)
    out[...] = (jnp.dot(t, k3w[...], preferred_element_type=jnp.float32)
                + k3b[...]).astype(jnp.bfloat16)


def _kform(ea, k1w, k1b, k2w, k2b, k3w, k3b):
    n_blk = _E // _KF_BLK
    full = lambda *shape: pl.BlockSpec(shape, lambda i, s=len(shape): (0,) * s)
    return pl.pallas_call(
        _kform_body,
        grid=(n_blk,),
        in_specs=[
            pl.BlockSpec((_KF_BLK, 16), lambda i: (i, 0)),
            full(16, 256), full(1, 256),
            full(256, 512), full(1, 512),
            full(512, 4096), full(1, 4096),
        ],
        out_specs=pl.BlockSpec((_KF_BLK, 4096), lambda i: (i, 0)),
        out_shape=jax.ShapeDtypeStruct((_E, 4096), jnp.bfloat16),
    )(ea, k1w, k1b, k2w, k2b, k3w, k3b)


def _matvec_body(xj, K, B, out):
    for eb in range(_MV_BLK // _EB):
        x = xj[pl.ds(eb * _EB, _EB), :]
        acc = jnp.zeros((_EB, 2 * _W), jnp.float32)
        for i in range(_W // 2):
            xi = jnp.dot(x, B[:, 128 * i:128 * (i + 1)],
                         preferred_element_type=jnp.float32)
            kk = K[pl.ds(eb * _EB, _EB), 128 * i:128 * (i + 1)].astype(jnp.float32)
            acc = acc + xi * kk
        out[pl.ds(eb * _EB, _EB), :] = acc[:, :_W] + acc[:, _W:]


def _matvec(xj, K):
    n_blk = _E // _MV_BLK
    return pl.pallas_call(
        _matvec_body,
        grid=(n_blk,),
        in_specs=[
            pl.BlockSpec((_MV_BLK, _W), lambda i: (i, 0)),
            pl.BlockSpec((_MV_BLK, 4096), lambda i: (i, 0)),
            pl.BlockSpec((_W, 4096), lambda i: (0, 0)),
        ],
        out_specs=pl.BlockSpec((_MV_BLK, _W), lambda i: (i, 0)),
        out_shape=jax.ShapeDtypeStruct((_E, _W), jnp.float32),
    )(xj, K, jnp.asarray(_BX))


def _update_body(a0, a1, d0, d1, h, rootw, rootb, out):
    hr = jnp.dot(h[...], rootw[...], preferred_element_type=jnp.float32) + rootb[...]
    deg = jnp.maximum(d0[...] + d1[...], 1.0)
    out[...] = jnp.maximum((a0[...] + a1[...]) / deg + hr, 0.0)


def _update(a0, a1, d0, d1, h, rootw, rootb):
    n_blk = _N // _ND_BLK
    nd = pl.BlockSpec((_ND_BLK, _W), lambda i: (i, 0))
    return pl.pallas_call(
        _update_body,
        grid=(n_blk,),
        in_specs=[
            nd, nd, nd, nd, nd,
            pl.BlockSpec((_W, _W), lambda i: (0, 0)),
            pl.BlockSpec((1, _W), lambda i: (0, 0)),
        ],
        out_specs=pl.BlockSpec((_ND_BLK, _W), lambda i: (i, 0)),
        out_shape=jax.ShapeDtypeStruct((_N, _W), jnp.float32),
    )(a0, a1, d0, d1, h, rootw, rootb)


def _h0_body(x, w, b, out):
    out[...] = x[...] * w[...] + b[...]


def _h0(x, w, b):
    n_blk = _N // _ND_BLK
    return pl.pallas_call(
        _h0_body,
        grid=(n_blk,),
        in_specs=[
            pl.BlockSpec((_ND_BLK, 1), lambda i: (i, 0)),
            pl.BlockSpec((1, _W), lambda i: (0, 0)),
            pl.BlockSpec((1, _W), lambda i: (0, 0)),
        ],
        out_specs=pl.BlockSpec((_ND_BLK, _W), lambda i: (i, 0)),
        out_shape=jax.ShapeDtypeStruct((_N, _W), jnp.float32),
    )(x, w, b)


def _fc2_body(h, w, b, out):
    out[...] = jnp.sum(h[...] * w[...], axis=1, keepdims=True) + b[...]


def _fc2(h, w, b):
    n_blk = _N // _ND_BLK
    return pl.pallas_call(
        _fc2_body,
        grid=(n_blk,),
        in_specs=[
            pl.BlockSpec((_ND_BLK, _W), lambda i: (i, 0)),
            pl.BlockSpec((1, _W), lambda i: (0, 0)),
            pl.BlockSpec((1, 1), lambda i: (0, 0)),
        ],
        out_specs=pl.BlockSpec((_ND_BLK, 1), lambda i: (i, 0)),
        out_shape=jax.ShapeDtypeStruct((_N, 1), jnp.float32),
    )(h, w, b)


def _gather_body(h_hbm, src_hbm, out_hbm, idx_f, idx_t, rows_v, sem):
    wid = lax.axis_index("s") * _NC + lax.axis_index("c")
    base = wid * _EPW

    def chunk(j, _):
        off = base + j * _GCH
        pltpu.sync_copy(src_hbm.at[pl.ds(off, _GCH)], idx_f)
        pltpu.async_copy(h_hbm.at[idx_f], rows_v, sem).wait()
        pltpu.sync_copy(rows_v, out_hbm.at[pl.ds(off, _GCH)])
        return 0

    lax.fori_loop(0, _NFULL, chunk, 0)
    off = base + _NFULL * _GCH
    pltpu.sync_copy(src_hbm.at[pl.ds(off, _TAIL)], idx_t)
    pltpu.async_copy(h_hbm.at[idx_t], rows_v.at[pl.ds(0, _TAIL)], sem).wait()
    pltpu.sync_copy(rows_v.at[pl.ds(0, _TAIL)], out_hbm.at[pl.ds(off, _TAIL)])


def _scatter_body(msg_hbm, dst_hbm, zeros_hbm, out_hbm, idx_f, idx_t, msg_v,
                  agg_sh):
    c = lax.axis_index("c")
    s = lax.axis_index("s")
    wid = s * _NC + c
    pltpu.sync_copy(zeros_hbm.at[pl.ds(s * _RPT, _RPT)],
                    agg_sh.at[pl.ds(s * _RPT, _RPT)])
    plsc.subcore_barrier()
    base = wid * _EPW

    def chunk(j, _):
        off = base + j * _GCH
        pltpu.sync_copy(dst_hbm.at[pl.ds(off, _GCH)], idx_f)
        pltpu.sync_copy(msg_hbm.at[pl.ds(off, _GCH)], msg_v)
        pltpu.sync_copy(msg_v, agg_sh.at[idx_f], add=True)
        return 0

    lax.fori_loop(0, _NFULL, chunk, 0)
    off = base + _NFULL * _GCH
    pltpu.sync_copy(dst_hbm.at[pl.ds(off, _TAIL)], idx_t)
    pltpu.sync_copy(msg_hbm.at[pl.ds(off, _TAIL)], msg_v.at[pl.ds(0, _TAIL)])
    pltpu.sync_copy(msg_v.at[pl.ds(0, _TAIL)], agg_sh.at[idx_t], add=True)
    plsc.subcore_barrier()
    pltpu.sync_copy(agg_sh.at[pl.ds(s * _RPT, _RPT)],
                    out_hbm.at[c, pl.ds(s * _RPT, _RPT)])


@functools.cache
def _sc_kernels():
    mesh = plsc.VectorSubcoreMesh(core_axis_name="c", subcore_axis_name="s",
                                  num_cores=_NC, num_subcores=_NS)
    gather = pl.kernel(
        _gather_body,
        out_type=jax.ShapeDtypeStruct((_E, _W), jnp.float32),
        mesh=mesh,
        compiler_params=pltpu.CompilerParams(use_tc_tiling_on_sc=False),
        scratch_types=[
            pltpu.VMEM((_GCH,), jnp.int32),
            pltpu.VMEM((_TAIL,), jnp.int32),
            pltpu.VMEM((_GCH, _W), jnp.float32),
            pltpu.SemaphoreType.DMA,
        ],
    )
    scatter = pl.kernel(
        _scatter_body,
        out_type=jax.ShapeDtypeStruct((_NC, _N, _W), jnp.float32),
        mesh=mesh,
        compiler_params=pltpu.CompilerParams(use_tc_tiling_on_sc=False),
        scratch_types=[
            pltpu.VMEM((_GCH,), jnp.int32),
            pltpu.VMEM((_TAIL,), jnp.int32),
            pltpu.VMEM((_GCH, _W), jnp.float32),
            pltpu.VMEM_SHARED((_N, _W), jnp.float32),
        ],
    )
    return gather, scatter


def kernel(x, edge_index, edge_attr, fc1_W, fc1_b, k1_W, k1_b, k2_W, k2_b,
           k3_W, k3_b, root_W, root_b, fc2_W, fc2_b):
    src = edge_index[0]
    dst = edge_index[1]

    K = _kform(edge_attr, k1_W, k1_b.reshape(1, -1), k2_W, k2_b.reshape(1, -1),
               k3_W[:, _PERM], k3_b[_PERM].reshape(1, -1))

    h = _h0(x, fc1_W, fc1_b.reshape(1, -1))

    _sc_gather, _sc_scatter = _sc_kernels()
    zeros_nw = jnp.zeros((_N, _W), jnp.float32)
    ones_ew = jnp.ones((_E, _W), jnp.float32)
    degp = _sc_scatter(ones_ew, dst, zeros_nw)
    d0, d1 = degp[0], degp[1]

    for _ in range(_DEPTH):
        xj = _sc_gather(h, src)
        msg = _matvec(xj, K)
        aggp = _sc_scatter(msg, dst, zeros_nw)
        h = _update(aggp[0], aggp[1], d0, d1, h, root_W, root_b.reshape(1, -1))

    return _fc2(h, fc2_W.reshape(1, _W), fc2_b.reshape(1, 1))


# speedup vs baseline: 4.9266x; 1.0560x over previous
import functools

import jax
import jax.numpy as jnp
import numpy as np
from jax import lax
from jax.experimental import pallas as pl
from jax.experimental.pallas import tpu as pltpu
from jax.experimental.pallas import tpu_sc as plsc

_N = 10000
_E = 160000
_W = 64
_DEPTH = 3

_NC = 2
_NS = 16
_NW = _NC * _NS
_EPW = _E // _NW
_GCH = 128
_NFULL = _EPW // _GCH
_TAIL = _EPW - _NFULL * _GCH
_RPT = _N // _NS
_SCH = 512
_SUP = [(j * _SCH, _SCH) for j in range(9)] + [(9 * _SCH, _EPW - 9 * _SCH)]


def _subchunks(size):
    subs = []
    off = 0
    while off < size:
        n = min(_GCH, size - off)
        subs.append((off, n))
        off += n
    return subs

_PERM = np.empty((4096,), np.int32)
for _i in range(32):
    for _p in range(2):
        for _g in range(64):
            _PERM[128 * _i + 64 * _p + _g] = (2 * _i + _p) * 64 + _g

_BX = np.zeros((64, 4096), np.float32)
for _i in range(32):
    for _p in range(2):
        _BX[2 * _i + _p, 128 * _i + 64 * _p:128 * _i + 64 * _p + 64] = 1.0

_EB = 128

_KF_BLK = 256
_MV_BLK = 640
_ND_BLK = 1000


def _kform_body(ea, k1w, k1b, k2w, k2b, k3w, k3b, out):
    t = jnp.maximum(jnp.dot(ea[...], k1w[...], preferred_element_type=jnp.float32) + k1b[...], 0.0)
    t = jnp.maximum(jnp.dot(t, k2w[...], preferred_element_type=jnp.float32) + k2b[...], 0.0)
    out[...] = (jnp.dot(t, k3w[...], preferred_element_type=jnp.float32)
                + k3b[...]).astype(jnp.bfloat16)


def _kform(ea, k1w, k1b, k2w, k2b, k3w, k3b):
    n_blk = _E // _KF_BLK
    full = lambda *shape: pl.BlockSpec(shape, lambda i, s=len(shape): (0,) * s)
    return pl.pallas_call(
        _kform_body,
        grid=(n_blk,),
        in_specs=[
            pl.BlockSpec((_KF_BLK, 16), lambda i: (i, 0)),
            full(16, 256), full(1, 256),
            full(256, 512), full(1, 512),
            full(512, 4096), full(1, 4096),
        ],
        out_specs=pl.BlockSpec((_KF_BLK, 4096), lambda i: (i, 0)),
        out_shape=jax.ShapeDtypeStruct((_E, 4096), jnp.bfloat16),
    )(ea, k1w, k1b, k2w, k2b, k3w, k3b)


def _matvec_body(xj, K, B, out):
    for eb in range(_MV_BLK // _EB):
        x = xj[pl.ds(eb * _EB, _EB), :]
        acc = jnp.zeros((_EB, 2 * _W), jnp.float32)
        for i in range(_W // 2):
            xi = jnp.dot(x, B[:, 128 * i:128 * (i + 1)],
                         preferred_element_type=jnp.float32)
            kk = K[pl.ds(eb * _EB, _EB), 128 * i:128 * (i + 1)].astype(jnp.float32)
            acc = acc + xi * kk
        out[pl.ds(eb * _EB, _EB), :] = acc[:, :_W] + acc[:, _W:]


def _matvec(xj, K):
    n_blk = _E // _MV_BLK
    return pl.pallas_call(
        _matvec_body,
        grid=(n_blk,),
        in_specs=[
            pl.BlockSpec((_MV_BLK, _W), lambda i: (i, 0)),
            pl.BlockSpec((_MV_BLK, 4096), lambda i: (i, 0)),
            pl.BlockSpec((_W, 4096), lambda i: (0, 0)),
        ],
        out_specs=pl.BlockSpec((_MV_BLK, _W), lambda i: (i, 0)),
        out_shape=jax.ShapeDtypeStruct((_E, _W), jnp.float32),
    )(xj, K, jnp.asarray(_BX))


def _update_body(a0, a1, d0, d1, h, rootw, rootb, out):
    hr = jnp.dot(h[...], rootw[...], preferred_element_type=jnp.float32) + rootb[...]
    deg = jnp.maximum(d0[...] + d1[...], 1.0)
    out[...] = jnp.maximum((a0[...] + a1[...]) / deg + hr, 0.0)


def _update(a0, a1, d0, d1, h, rootw, rootb):
    n_blk = _N // _ND_BLK
    nd = pl.BlockSpec((_ND_BLK, _W), lambda i: (i, 0))
    return pl.pallas_call(
        _update_body,
        grid=(n_blk,),
        in_specs=[
            nd, nd, nd, nd, nd,
            pl.BlockSpec((_W, _W), lambda i: (0, 0)),
            pl.BlockSpec((1, _W), lambda i: (0, 0)),
        ],
        out_specs=pl.BlockSpec((_ND_BLK, _W), lambda i: (i, 0)),
        out_shape=jax.ShapeDtypeStruct((_N, _W), jnp.float32),
    )(a0, a1, d0, d1, h, rootw, rootb)


def _h0_body(x, w, b, out):
    out[...] = x[...] * w[...] + b[...]


def _h0(x, w, b):
    n_blk = _N // _ND_BLK
    return pl.pallas_call(
        _h0_body,
        grid=(n_blk,),
        in_specs=[
            pl.BlockSpec((_ND_BLK, 1), lambda i: (i, 0)),
            pl.BlockSpec((1, _W), lambda i: (0, 0)),
            pl.BlockSpec((1, _W), lambda i: (0, 0)),
        ],
        out_specs=pl.BlockSpec((_ND_BLK, _W), lambda i: (i, 0)),
        out_shape=jax.ShapeDtypeStruct((_N, _W), jnp.float32),
    )(x, w, b)


def _fc2_body(h, w, b, out):
    out[...] = jnp.sum(h[...] * w[...], axis=1, keepdims=True) + b[...]


def _fc2(h, w, b):
    n_blk = _N // _ND_BLK
    return pl.pallas_call(
        _fc2_body,
        grid=(n_blk,),
        in_specs=[
            pl.BlockSpec((_ND_BLK, _W), lambda i: (i, 0)),
            pl.BlockSpec((1, _W), lambda i: (0, 0)),
            pl.BlockSpec((1, 1), lambda i: (0, 0)),
        ],
        out_specs=pl.BlockSpec((_ND_BLK, 1), lambda i: (i, 0)),
        out_shape=jax.ShapeDtypeStruct((_N, 1), jnp.float32),
    )(h, w, b)


def _gather_body(h_hbm, src_hbm, out_hbm, idx3, idxt, rows2, sem_i, sem_g,
                 sem_s0, sem_s1):
    wid = lax.axis_index("s") * _NC + lax.axis_index("c")
    base = wid * _EPW
    ssem = (sem_s0, sem_s1)
    idx_descs = {}
    store_descs = {}

    def issue_idx(j):
        b = j % 2
        off0, size = _SUP[j]
        ds_ = []
        for k, (soff, n) in enumerate(_subchunks(size)):
            dst = idx3.at[b, k] if n == _GCH else idxt
            ds_.append(pltpu.async_copy(
                src_hbm.at[pl.ds(base + off0 + soff, n)], dst, sem_i))
        idx_descs[j] = ds_

    issue_idx(0)
    for j in range(len(_SUP)):
        b = j % 2
        off0, size = _SUP[j]
        if j - 2 in store_descs:
            store_descs.pop(j - 2).wait()
        for d in idx_descs.pop(j):
            d.wait()
        if j + 1 < len(_SUP):
            issue_idx(j + 1)
        g_descs = []
        for k, (soff, n) in enumerate(_subchunks(size)):
            idx = idx3.at[b, k] if n == _GCH else idxt
            g_descs.append(pltpu.async_copy(
                h_hbm.at[idx], rows2.at[b, pl.ds(soff, n)], sem_g))
        for d in g_descs:
            d.wait()
        store_descs[j] = pltpu.async_copy(
            rows2.at[b, pl.ds(0, size)], out_hbm.at[pl.ds(base + off0, size)],
            ssem[b])
    for j in sorted(store_descs):
        store_descs[j].wait()


def _scatter_body(msg_hbm, dst_hbm, zeros_hbm, out_hbm, idx3, idxt, msg2,
                  agg_sh, sem_i, sem_m, sem_a):
    c = lax.axis_index("c")
    s = lax.axis_index("s")
    wid = s * _NC + c
    base = wid * _EPW
    idx_descs = {}
    msg_descs = {}
    add_descs = {}

    def issue_loads(j):
        b = j % 2
        off0, size = _SUP[j]
        di, dm = [], []
        for k, (soff, n) in enumerate(_subchunks(size)):
            dst = idx3.at[b, k] if n == _GCH else idxt
            di.append(pltpu.async_copy(
                dst_hbm.at[pl.ds(base + off0 + soff, n)], dst, sem_i))
        dm.append(pltpu.async_copy(
            msg_hbm.at[pl.ds(base + off0, size)], msg2.at[b, pl.ds(0, size)],
            sem_m))
        idx_descs[j] = di
        msg_descs[j] = dm

    issue_loads(0)
    pltpu.sync_copy(zeros_hbm.at[pl.ds(s * _RPT, _RPT)],
                    agg_sh.at[pl.ds(s * _RPT, _RPT)])
    plsc.subcore_barrier()
    for j in range(len(_SUP)):
        b = j % 2
        off0, size = _SUP[j]
        if j - 1 in add_descs:
            for d in add_descs.pop(j - 1):
                d.wait()
        for d in idx_descs.pop(j) + msg_descs.pop(j):
            d.wait()
        if j + 1 < len(_SUP):
            issue_loads(j + 1)
        ads = []
        for k, (soff, n) in enumerate(_subchunks(size)):
            idx = idx3.at[b, k] if n == _GCH else idxt
            ads.append(pltpu.async_copy(
                msg2.at[b, pl.ds(soff, n)], agg_sh.at[idx], sem_a, add=True))
        add_descs[j] = ads
    for j in sorted(add_descs):
        for d in add_descs.pop(j):
            d.wait()
    plsc.subcore_barrier()
    pltpu.sync_copy(agg_sh.at[pl.ds(s * _RPT, _RPT)],
                    out_hbm.at[c, pl.ds(s * _RPT, _RPT)])


@functools.cache
def _sc_kernels():
    mesh = plsc.VectorSubcoreMesh(core_axis_name="c", subcore_axis_name="s",
                                  num_cores=_NC, num_subcores=_NS)
    nsub = _EPW // _SCH + 1
    gather = pl.kernel(
        _gather_body,
        out_type=jax.ShapeDtypeStruct((_E, _W), jnp.float32),
        mesh=mesh,
        compiler_params=pltpu.CompilerParams(use_tc_tiling_on_sc=False),
        scratch_types=[
            pltpu.VMEM((2, _SCH // _GCH, _GCH), jnp.int32),
            pltpu.VMEM((_TAIL,), jnp.int32),
            pltpu.VMEM((2, _SCH, _W), jnp.float32),
            pltpu.SemaphoreType.DMA,
            pltpu.SemaphoreType.DMA,
            pltpu.SemaphoreType.DMA,
            pltpu.SemaphoreType.DMA,
        ],
    )
    scatter = pl.kernel(
        _scatter_body,
        out_type=jax.ShapeDtypeStruct((_NC, _N, _W), jnp.float32),
        mesh=mesh,
        compiler_params=pltpu.CompilerParams(use_tc_tiling_on_sc=False),
        scratch_types=[
            pltpu.VMEM((2, _SCH // _GCH, _GCH), jnp.int32),
            pltpu.VMEM((_TAIL,), jnp.int32),
            pltpu.VMEM((2, _SCH, _W), jnp.float32),
            pltpu.VMEM_SHARED((_N, _W), jnp.float32),
            pltpu.SemaphoreType.DMA,
            pltpu.SemaphoreType.DMA,
            pltpu.SemaphoreType.DMA,
        ],
    )
    return gather, scatter


def kernel(x, edge_index, edge_attr, fc1_W, fc1_b, k1_W, k1_b, k2_W, k2_b,
           k3_W, k3_b, root_W, root_b, fc2_W, fc2_b):
    src = edge_index[0]
    dst = edge_index[1]

    K = _kform(edge_attr, k1_W, k1_b.reshape(1, -1), k2_W, k2_b.reshape(1, -1),
               k3_W[:, _PERM], k3_b[_PERM].reshape(1, -1))

    h = _h0(x, fc1_W, fc1_b.reshape(1, -1))

    _sc_gather, _sc_scatter = _sc_kernels()
    zeros_nw = jnp.zeros((_N, _W), jnp.float32)
    ones_ew = jnp.ones((_E, _W), jnp.float32)
    degp = _sc_scatter(ones_ew, dst, zeros_nw)
    d0, d1 = degp[0], degp[1]

    for _ in range(_DEPTH):
        xj = _sc_gather(h, src)
        msg = _matvec(xj, K)
        aggp = _sc_scatter(msg, dst, zeros_nw)
        h = _update(aggp[0], aggp[1], d0, d1, h, root_W, root_b.reshape(1, -1))

    return _fc2(h, fc2_W.reshape(1, _W), fc2_b.reshape(1, 1))


# speedup vs baseline: 5.5915x; 1.1350x over previous
import functools

import jax
import jax.numpy as jnp
import numpy as np
from jax import lax
from jax.experimental import pallas as pl
from jax.experimental.pallas import tpu as pltpu
from jax.experimental.pallas import tpu_sc as plsc

_N = 10000
_E = 160000
_W = 64
_DEPTH = 3

_NC = 2
_NS = 16
_NW = _NC * _NS
_EPW = _E // _NW
_GCH = 128
_NFULL = _EPW // _GCH
_TAIL = _EPW - _NFULL * _GCH
_RPT = _N // _NS
_SCH = 512
_SUP = [(j * _SCH, _SCH) for j in range(9)] + [(9 * _SCH, _EPW - 9 * _SCH)]


def _subchunks(size):
    subs = []
    off = 0
    while off < size:
        n = min(_GCH, size - off)
        subs.append((off, n))
        off += n
    return subs

_PERM = np.empty((4096,), np.int32)
for _i in range(32):
    for _p in range(2):
        for _g in range(64):
            _PERM[128 * _i + 64 * _p + _g] = (2 * _i + _p) * 64 + _g

_BX = np.zeros((64, 4096), np.float32)
for _i in range(32):
    for _p in range(2):
        _BX[2 * _i + _p, 128 * _i + 64 * _p:128 * _i + 64 * _p + 64] = 1.0

_EB = 128

_KF_BLK = 640
_MV_BLK = 1280
_ND_BLK = 1000


def _kform_body(ea, k1w, k1b, k2w, k2b, k3w, k3b, out):
    t = jnp.maximum(jnp.dot(ea[...], k1w[...], preferred_element_type=jnp.float32) + k1b[...], 0.0)
    t = jnp.maximum(jnp.dot(t, k2w[...], preferred_element_type=jnp.float32) + k2b[...], 0.0)
    out[...] = (jnp.dot(t.astype(jnp.bfloat16), k3w[...],
                        preferred_element_type=jnp.float32)
                + k3b[...]).astype(jnp.bfloat16)


def _kform(ea, k1w, k1b, k2w, k2b, k3w, k3b):
    n_blk = _E // _KF_BLK
    full = lambda *shape: pl.BlockSpec(shape, lambda i, s=len(shape): (0,) * s)
    return pl.pallas_call(
        _kform_body,
        grid=(n_blk,),
        in_specs=[
            pl.BlockSpec((_KF_BLK, 16), lambda i: (i, 0)),
            full(16, 256), full(1, 256),
            full(256, 512), full(1, 512),
            full(512, 4096), full(1, 4096),
        ],
        out_specs=pl.BlockSpec((_KF_BLK, 4096), lambda i: (i, 0)),
        out_shape=jax.ShapeDtypeStruct((_E, 4096), jnp.bfloat16),
    )(ea, k1w, k1b, k2w, k2b, k3w, k3b)


def _matvec_body(xj, K, B, out):
    for eb in range(_MV_BLK // _EB):
        x = xj[pl.ds(eb * _EB, _EB), :]
        acc = jnp.zeros((_EB, 2 * _W), jnp.float32)
        for i in range(_W // 2):
            xi = jnp.dot(x, B[:, 128 * i:128 * (i + 1)],
                         preferred_element_type=jnp.float32)
            kk = K[pl.ds(eb * _EB, _EB), 128 * i:128 * (i + 1)].astype(jnp.float32)
            acc = acc + xi * kk
        out[pl.ds(eb * _EB, _EB), :] = acc[:, :_W] + acc[:, _W:]


def _matvec(xj, K):
    n_blk = _E // _MV_BLK
    return pl.pallas_call(
        _matvec_body,
        grid=(n_blk,),
        in_specs=[
            pl.BlockSpec((_MV_BLK, _W), lambda i: (i, 0)),
            pl.BlockSpec((_MV_BLK, 4096), lambda i: (i, 0)),
            pl.BlockSpec((_W, 4096), lambda i: (0, 0)),
        ],
        out_specs=pl.BlockSpec((_MV_BLK, _W), lambda i: (i, 0)),
        out_shape=jax.ShapeDtypeStruct((_E, _W), jnp.float32),
    )(xj, K, jnp.asarray(_BX))


def _update_body(a0, a1, d0, d1, h, rootw, rootb, out):
    hr = jnp.dot(h[...], rootw[...], preferred_element_type=jnp.float32) + rootb[...]
    deg = jnp.maximum(d0[...] + d1[...], 1.0)
    out[...] = jnp.maximum((a0[...] + a1[...]) / deg + hr, 0.0)


def _update(a0, a1, d0, d1, h, rootw, rootb):
    n_blk = _N // _ND_BLK
    nd = pl.BlockSpec((_ND_BLK, _W), lambda i: (i, 0))
    return pl.pallas_call(
        _update_body,
        grid=(n_blk,),
        in_specs=[
            nd, nd, nd, nd, nd,
            pl.BlockSpec((_W, _W), lambda i: (0, 0)),
            pl.BlockSpec((1, _W), lambda i: (0, 0)),
        ],
        out_specs=pl.BlockSpec((_ND_BLK, _W), lambda i: (i, 0)),
        out_shape=jax.ShapeDtypeStruct((_N, _W), jnp.float32),
    )(a0, a1, d0, d1, h, rootw, rootb)


def _h0_body(x, w, b, out):
    out[...] = x[...] * w[...] + b[...]


def _h0(x, w, b):
    n_blk = _N // _ND_BLK
    return pl.pallas_call(
        _h0_body,
        grid=(n_blk,),
        in_specs=[
            pl.BlockSpec((_ND_BLK, 1), lambda i: (i, 0)),
            pl.BlockSpec((1, _W), lambda i: (0, 0)),
            pl.BlockSpec((1, _W), lambda i: (0, 0)),
        ],
        out_specs=pl.BlockSpec((_ND_BLK, _W), lambda i: (i, 0)),
        out_shape=jax.ShapeDtypeStruct((_N, _W), jnp.float32),
    )(x, w, b)


def _fc2_body(h, w, b, out):
    out[...] = jnp.sum(h[...] * w[...], axis=1, keepdims=True) + b[...]


def _fc2(h, w, b):
    n_blk = _N // _ND_BLK
    return pl.pallas_call(
        _fc2_body,
        grid=(n_blk,),
        in_specs=[
            pl.BlockSpec((_ND_BLK, _W), lambda i: (i, 0)),
            pl.BlockSpec((1, _W), lambda i: (0, 0)),
            pl.BlockSpec((1, 1), lambda i: (0, 0)),
        ],
        out_specs=pl.BlockSpec((_ND_BLK, 1), lambda i: (i, 0)),
        out_shape=jax.ShapeDtypeStruct((_N, 1), jnp.float32),
    )(h, w, b)


def _gather_body(h_hbm, src_hbm, out_hbm, idx3, idxt, rows2, sem_i, sem_g,
                 sem_s0, sem_s1):
    wid = lax.axis_index("s") * _NC + lax.axis_index("c")
    base = wid * _EPW
    ssem = (sem_s0, sem_s1)
    idx_descs = {}
    store_descs = {}

    def issue_idx(j):
        b = j % 2
        off0, size = _SUP[j]
        ds_ = []
        for k, (soff, n) in enumerate(_subchunks(size)):
            dst = idx3.at[b, k] if n == _GCH else idxt
            ds_.append(pltpu.async_copy(
                src_hbm.at[pl.ds(base + off0 + soff, n)], dst, sem_i))
        idx_descs[j] = ds_

    issue_idx(0)
    for j in range(len(_SUP)):
        b = j % 2
        off0, size = _SUP[j]
        if j - 2 in store_descs:
            store_descs.pop(j - 2).wait()
        for d in idx_descs.pop(j):
            d.wait()
        if j + 1 < len(_SUP):
            issue_idx(j + 1)
        g_descs = []
        for k, (soff, n) in enumerate(_subchunks(size)):
            idx = idx3.at[b, k] if n == _GCH else idxt
            g_descs.append(pltpu.async_copy(
                h_hbm.at[idx], rows2.at[b, pl.ds(soff, n)], sem_g))
        for d in g_descs:
            d.wait()
        store_descs[j] = pltpu.async_copy(
            rows2.at[b, pl.ds(0, size)], out_hbm.at[pl.ds(base + off0, size)],
            ssem[b])
    for j in sorted(store_descs):
        store_descs[j].wait()


def _scatter_body(msg_hbm, dst_hbm, zeros_hbm, out_hbm, idx3, idxt, msg2,
                  agg_sh, sem_i, sem_m, sem_a):
    c = lax.axis_index("c")
    s = lax.axis_index("s")
    wid = s * _NC + c
    base = wid * _EPW
    idx_descs = {}
    msg_descs = {}
    add_descs = {}

    def issue_loads(j):
        b = j % 2
        off0, size = _SUP[j]
        di, dm = [], []
        for k, (soff, n) in enumerate(_subchunks(size)):
            dst = idx3.at[b, k] if n == _GCH else idxt
            di.append(pltpu.async_copy(
                dst_hbm.at[pl.ds(base + off0 + soff, n)], dst, sem_i))
        dm.append(pltpu.async_copy(
            msg_hbm.at[pl.ds(base + off0, size)], msg2.at[b, pl.ds(0, size)],
            sem_m))
        idx_descs[j] = di
        msg_descs[j] = dm

    issue_loads(0)
    pltpu.sync_copy(zeros_hbm.at[pl.ds(s * _RPT, _RPT)],
                    agg_sh.at[pl.ds(s * _RPT, _RPT)])
    plsc.subcore_barrier()
    for j in range(len(_SUP)):
        b = j % 2
        off0, size = _SUP[j]
        if j - 1 in add_descs:
            for d in add_descs.pop(j - 1):
                d.wait()
        for d in idx_descs.pop(j) + msg_descs.pop(j):
            d.wait()
        if j + 1 < len(_SUP):
            issue_loads(j + 1)
        ads = []
        for k, (soff, n) in enumerate(_subchunks(size)):
            idx = idx3.at[b, k] if n == _GCH else idxt
            ads.append(pltpu.async_copy(
                msg2.at[b, pl.ds(soff, n)], agg_sh.at[idx], sem_a, add=True))
        add_descs[j] = ads
    for j in sorted(add_descs):
        for d in add_descs.pop(j):
            d.wait()
    plsc.subcore_barrier()
    pltpu.sync_copy(agg_sh.at[pl.ds(s * _RPT, _RPT)],
                    out_hbm.at[c, pl.ds(s * _RPT, _RPT)])


@functools.cache
def _sc_kernels():
    mesh = plsc.VectorSubcoreMesh(core_axis_name="c", subcore_axis_name="s",
                                  num_cores=_NC, num_subcores=_NS)
    nsub = _EPW // _SCH + 1
    gather = pl.kernel(
        _gather_body,
        out_type=jax.ShapeDtypeStruct((_E, _W), jnp.float32),
        mesh=mesh,
        compiler_params=pltpu.CompilerParams(use_tc_tiling_on_sc=False),
        scratch_types=[
            pltpu.VMEM((2, _SCH // _GCH, _GCH), jnp.int32),
            pltpu.VMEM((_TAIL,), jnp.int32),
            pltpu.VMEM((2, _SCH, _W), jnp.float32),
            pltpu.SemaphoreType.DMA,
            pltpu.SemaphoreType.DMA,
            pltpu.SemaphoreType.DMA,
            pltpu.SemaphoreType.DMA,
        ],
    )
    scatter = pl.kernel(
        _scatter_body,
        out_type=jax.ShapeDtypeStruct((_NC, _N, _W), jnp.float32),
        mesh=mesh,
        compiler_params=pltpu.CompilerParams(use_tc_tiling_on_sc=False),
        scratch_types=[
            pltpu.VMEM((2, _SCH // _GCH, _GCH), jnp.int32),
            pltpu.VMEM((_TAIL,), jnp.int32),
            pltpu.VMEM((2, _SCH, _W), jnp.float32),
            pltpu.VMEM_SHARED((_N, _W), jnp.float32),
            pltpu.SemaphoreType.DMA,
            pltpu.SemaphoreType.DMA,
            pltpu.SemaphoreType.DMA,
        ],
    )
    return gather, scatter


def kernel(x, edge_index, edge_attr, fc1_W, fc1_b, k1_W, k1_b, k2_W, k2_b,
           k3_W, k3_b, root_W, root_b, fc2_W, fc2_b):
    src = edge_index[0]
    dst = edge_index[1]

    K = _kform(edge_attr, k1_W, k1_b.reshape(1, -1), k2_W, k2_b.reshape(1, -1),
               k3_W[:, _PERM].astype(jnp.bfloat16), k3_b[_PERM].reshape(1, -1))

    h = _h0(x, fc1_W, fc1_b.reshape(1, -1))

    _sc_gather, _sc_scatter = _sc_kernels()
    zeros_nw = jnp.zeros((_N, _W), jnp.float32)
    ones_ew = jnp.ones((_E, _W), jnp.float32)
    degp = _sc_scatter(ones_ew, dst, zeros_nw)
    d0, d1 = degp[0], degp[1]

    for _ in range(_DEPTH):
        xj = _sc_gather(h, src)
        msg = _matvec(xj, K)
        aggp = _sc_scatter(msg, dst, zeros_nw)
        h = _update(aggp[0], aggp[1], d0, d1, h, root_W, root_b.reshape(1, -1))

    return _fc2(h, fc2_W.reshape(1, _W), fc2_b.reshape(1, 1))


# speedup vs baseline: 5.7686x; 1.0317x over previous
import functools

import jax
import jax.numpy as jnp
import numpy as np
from jax import lax
from jax.experimental import pallas as pl
from jax.experimental.pallas import tpu as pltpu
from jax.experimental.pallas import tpu_sc as plsc

_N = 10000
_E = 160000
_W = 64
_DEPTH = 3

_NC = 2
_NS = 16
_NW = _NC * _NS
_EPW = _E // _NW
_GCH = 128
_NFULL = _EPW // _GCH
_TAIL = _EPW - _NFULL * _GCH
_RPT = _N // _NS
_SCH = 512
_SUP = [(j * _SCH, _SCH) for j in range(9)] + [(9 * _SCH, _EPW - 9 * _SCH)]


def _subchunks(size):
    subs = []
    off = 0
    while off < size:
        n = min(_GCH, size - off)
        subs.append((off, n))
        off += n
    return subs

_PERM = np.empty((4096,), np.int32)
for _i in range(32):
    for _p in range(2):
        for _g in range(64):
            _PERM[128 * _i + 64 * _p + _g] = (2 * _i + _p) * 64 + _g

_BX = np.zeros((64, 4096), np.float32)
for _i in range(32):
    for _p in range(2):
        _BX[2 * _i + _p, 128 * _i + 64 * _p:128 * _i + 64 * _p + 64] = 1.0

_EB = 128

_KF_BLK = 640
_MV_BLK = 1280
_ND_BLK = 1000


def _kform_body(ea, k1w, k1b, k2w, k2b, k3w, k3b, xj, B, outk, outm):
    t = jnp.maximum(jnp.dot(ea[...], k1w[...], preferred_element_type=jnp.float32) + k1b[...], 0.0)
    t = jnp.maximum(jnp.dot(t, k2w[...], preferred_element_type=jnp.float32) + k2b[...], 0.0)
    Kf = jnp.dot(t.astype(jnp.bfloat16), k3w[...],
                 preferred_element_type=jnp.float32) + k3b[...]
    outk[...] = Kf.astype(jnp.bfloat16)
    for eb in range(_KF_BLK // _EB):
        x = xj[pl.ds(eb * _EB, _EB), :]
        acc = jnp.zeros((_EB, 2 * _W), jnp.float32)
        for i in range(_W // 2):
            xi = jnp.dot(x, B[:, 128 * i:128 * (i + 1)],
                         preferred_element_type=jnp.float32)
            acc = acc + xi * Kf[eb * _EB:(eb + 1) * _EB, 128 * i:128 * (i + 1)]
        outm[pl.ds(eb * _EB, _EB), :] = acc[:, :_W] + acc[:, _W:]


def _kform(ea, k1w, k1b, k2w, k2b, k3w, k3b, xj):
    n_blk = _E // _KF_BLK
    full = lambda *shape: pl.BlockSpec(shape, lambda i, s=len(shape): (0,) * s)
    return pl.pallas_call(
        _kform_body,
        grid=(n_blk,),
        in_specs=[
            pl.BlockSpec((_KF_BLK, 16), lambda i: (i, 0)),
            full(16, 256), full(1, 256),
            full(256, 512), full(1, 512),
            full(512, 4096), full(1, 4096),
            pl.BlockSpec((_KF_BLK, _W), lambda i: (i, 0)),
            full(_W, 4096),
        ],
        out_specs=(pl.BlockSpec((_KF_BLK, 4096), lambda i: (i, 0)),
                   pl.BlockSpec((_KF_BLK, _W), lambda i: (i, 0))),
        out_shape=(jax.ShapeDtypeStruct((_E, 4096), jnp.bfloat16),
                   jax.ShapeDtypeStruct((_E, _W), jnp.float32)),
    )(ea, k1w, k1b, k2w, k2b, k3w, k3b, xj, jnp.asarray(_BX))


def _matvec_body(xj, K, B, out):
    for eb in range(_MV_BLK // _EB):
        x = xj[pl.ds(eb * _EB, _EB), :]
        acc = jnp.zeros((_EB, 2 * _W), jnp.float32)
        for i in range(_W // 2):
            xi = jnp.dot(x, B[:, 128 * i:128 * (i + 1)],
                         preferred_element_type=jnp.float32)
            kk = K[pl.ds(eb * _EB, _EB), 128 * i:128 * (i + 1)].astype(jnp.float32)
            acc = acc + xi * kk
        out[pl.ds(eb * _EB, _EB), :] = acc[:, :_W] + acc[:, _W:]


def _matvec(xj, K):
    n_blk = _E // _MV_BLK
    return pl.pallas_call(
        _matvec_body,
        grid=(n_blk,),
        in_specs=[
            pl.BlockSpec((_MV_BLK, _W), lambda i: (i, 0)),
            pl.BlockSpec((_MV_BLK, 4096), lambda i: (i, 0)),
            pl.BlockSpec((_W, 4096), lambda i: (0, 0)),
        ],
        out_specs=pl.BlockSpec((_MV_BLK, _W), lambda i: (i, 0)),
        out_shape=jax.ShapeDtypeStruct((_E, _W), jnp.float32),
    )(xj, K, jnp.asarray(_BX))


def _update_body(a0, a1, d0, d1, h, rootw, rootb, out):
    hr = jnp.dot(h[...], rootw[...], preferred_element_type=jnp.float32) + rootb[...]
    deg = jnp.maximum(d0[...] + d1[...], 1.0)
    out[...] = jnp.maximum((a0[...] + a1[...]) / deg + hr, 0.0)


def _update(a0, a1, d0, d1, h, rootw, rootb):
    n_blk = _N // _ND_BLK
    nd = pl.BlockSpec((_ND_BLK, _W), lambda i: (i, 0))
    return pl.pallas_call(
        _update_body,
        grid=(n_blk,),
        in_specs=[
            nd, nd, nd, nd, nd,
            pl.BlockSpec((_W, _W), lambda i: (0, 0)),
            pl.BlockSpec((1, _W), lambda i: (0, 0)),
        ],
        out_specs=pl.BlockSpec((_ND_BLK, _W), lambda i: (i, 0)),
        out_shape=jax.ShapeDtypeStruct((_N, _W), jnp.float32),
    )(a0, a1, d0, d1, h, rootw, rootb)


def _h0_body(x, w, b, out):
    out[...] = x[...] * w[...] + b[...]


def _h0(x, w, b):
    n_blk = _N // _ND_BLK
    return pl.pallas_call(
        _h0_body,
        grid=(n_blk,),
        in_specs=[
            pl.BlockSpec((_ND_BLK, 1), lambda i: (i, 0)),
            pl.BlockSpec((1, _W), lambda i: (0, 0)),
            pl.BlockSpec((1, _W), lambda i: (0, 0)),
        ],
        out_specs=pl.BlockSpec((_ND_BLK, _W), lambda i: (i, 0)),
        out_shape=jax.ShapeDtypeStruct((_N, _W), jnp.float32),
    )(x, w, b)


def _fc2_body(h, w, b, out):
    out[...] = jnp.sum(h[...] * w[...], axis=1, keepdims=True) + b[...]


def _fc2(h, w, b):
    n_blk = _N // _ND_BLK
    return pl.pallas_call(
        _fc2_body,
        grid=(n_blk,),
        in_specs=[
            pl.BlockSpec((_ND_BLK, _W), lambda i: (i, 0)),
            pl.BlockSpec((1, _W), lambda i: (0, 0)),
            pl.BlockSpec((1, 1), lambda i: (0, 0)),
        ],
        out_specs=pl.BlockSpec((_ND_BLK, 1), lambda i: (i, 0)),
        out_shape=jax.ShapeDtypeStruct((_N, 1), jnp.float32),
    )(h, w, b)


def _gather_body(h_hbm, src_hbm, out_hbm, idx3, idxt, rows2, sem_i, sem_g,
                 sem_s0, sem_s1):
    wid = lax.axis_index("s") * _NC + lax.axis_index("c")
    base = wid * _EPW
    ssem = (sem_s0, sem_s1)
    idx_descs = {}
    store_descs = {}

    def issue_idx(j):
        b = j % 2
        off0, size = _SUP[j]
        ds_ = []
        for k, (soff, n) in enumerate(_subchunks(size)):
            dst = idx3.at[b, k] if n == _GCH else idxt
            ds_.append(pltpu.async_copy(
                src_hbm.at[pl.ds(base + off0 + soff, n)], dst, sem_i))
        idx_descs[j] = ds_

    issue_idx(0)
    for j in range(len(_SUP)):
        b = j % 2
        off0, size = _SUP[j]
        if j - 2 in store_descs:
            store_descs.pop(j - 2).wait()
        for d in idx_descs.pop(j):
            d.wait()
        if j + 1 < len(_SUP):
            issue_idx(j + 1)
        g_descs = []
        for k, (soff, n) in enumerate(_subchunks(size)):
            idx = idx3.at[b, k] if n == _GCH else idxt
            g_descs.append(pltpu.async_copy(
                h_hbm.at[idx], rows2.at[b, pl.ds(soff, n)], sem_g))
        for d in g_descs:
            d.wait()
        store_descs[j] = pltpu.async_copy(
            rows2.at[b, pl.ds(0, size)], out_hbm.at[pl.ds(base + off0, size)],
            ssem[b])
    for j in sorted(store_descs):
        store_descs[j].wait()


def _scatter_body(msg_hbm, dst_hbm, zeros_hbm, out_hbm, idx3, idxt, msg2,
                  agg_sh, sem_i, sem_m, sem_a):
    c = lax.axis_index("c")
    s = lax.axis_index("s")
    wid = s * _NC + c
    base = wid * _EPW
    idx_descs = {}
    msg_descs = {}
    add_descs = {}

    def issue_loads(j):
        b = j % 2
        off0, size = _SUP[j]
        di, dm = [], []
        for k, (soff, n) in enumerate(_subchunks(size)):
            dst = idx3.at[b, k] if n == _GCH else idxt
            di.append(pltpu.async_copy(
                dst_hbm.at[pl.ds(base + off0 + soff, n)], dst, sem_i))
        dm.append(pltpu.async_copy(
            msg_hbm.at[pl.ds(base + off0, size)], msg2.at[b, pl.ds(0, size)],
            sem_m))
        idx_descs[j] = di
        msg_descs[j] = dm

    issue_loads(0)
    pltpu.sync_copy(zeros_hbm.at[pl.ds(s * _RPT, _RPT)],
                    agg_sh.at[pl.ds(s * _RPT, _RPT)])
    plsc.subcore_barrier()
    for j in range(len(_SUP)):
        b = j % 2
        off0, size = _SUP[j]
        if j - 1 in add_descs:
            for d in add_descs.pop(j - 1):
                d.wait()
        for d in idx_descs.pop(j) + msg_descs.pop(j):
            d.wait()
        if j + 1 < len(_SUP):
            issue_loads(j + 1)
        ads = []
        for k, (soff, n) in enumerate(_subchunks(size)):
            idx = idx3.at[b, k] if n == _GCH else idxt
            ads.append(pltpu.async_copy(
                msg2.at[b, pl.ds(soff, n)], agg_sh.at[idx], sem_a, add=True))
        add_descs[j] = ads
    for j in sorted(add_descs):
        for d in add_descs.pop(j):
            d.wait()
    plsc.subcore_barrier()
    pltpu.sync_copy(agg_sh.at[pl.ds(s * _RPT, _RPT)],
                    out_hbm.at[c, pl.ds(s * _RPT, _RPT)])


@functools.cache
def _sc_kernels():
    mesh = plsc.VectorSubcoreMesh(core_axis_name="c", subcore_axis_name="s",
                                  num_cores=_NC, num_subcores=_NS)
    nsub = _EPW // _SCH + 1
    gather = pl.kernel(
        _gather_body,
        out_type=jax.ShapeDtypeStruct((_E, _W), jnp.float32),
        mesh=mesh,
        compiler_params=pltpu.CompilerParams(use_tc_tiling_on_sc=False),
        scratch_types=[
            pltpu.VMEM((2, _SCH // _GCH, _GCH), jnp.int32),
            pltpu.VMEM((_TAIL,), jnp.int32),
            pltpu.VMEM((2, _SCH, _W), jnp.float32),
            pltpu.SemaphoreType.DMA,
            pltpu.SemaphoreType.DMA,
            pltpu.SemaphoreType.DMA,
            pltpu.SemaphoreType.DMA,
        ],
    )
    scatter = pl.kernel(
        _scatter_body,
        out_type=jax.ShapeDtypeStruct((_NC, _N, _W), jnp.float32),
        mesh=mesh,
        compiler_params=pltpu.CompilerParams(use_tc_tiling_on_sc=False),
        scratch_types=[
            pltpu.VMEM((2, _SCH // _GCH, _GCH), jnp.int32),
            pltpu.VMEM((_TAIL,), jnp.int32),
            pltpu.VMEM((2, _SCH, _W), jnp.float32),
            pltpu.VMEM_SHARED((_N, _W), jnp.float32),
            pltpu.SemaphoreType.DMA,
            pltpu.SemaphoreType.DMA,
            pltpu.SemaphoreType.DMA,
        ],
    )
    return gather, scatter


def kernel(x, edge_index, edge_attr, fc1_W, fc1_b, k1_W, k1_b, k2_W, k2_b,
           k3_W, k3_b, root_W, root_b, fc2_W, fc2_b):
    src = edge_index[0]
    dst = edge_index[1]

    h = _h0(x, fc1_W, fc1_b.reshape(1, -1))

    _sc_gather, _sc_scatter = _sc_kernels()
    zeros_nw = jnp.zeros((_N, _W), jnp.float32)
    ones_ew = jnp.ones((_E, _W), jnp.float32)
    degp = _sc_scatter(ones_ew, dst, zeros_nw)
    d0, d1 = degp[0], degp[1]

    xj = _sc_gather(h, src)
    K, msg = _kform(edge_attr, k1_W, k1_b.reshape(1, -1), k2_W,
                    k2_b.reshape(1, -1), k3_W[:, _PERM].astype(jnp.bfloat16),
                    k3_b[_PERM].reshape(1, -1), xj)

    for layer in range(_DEPTH):
        aggp = _sc_scatter(msg, dst, zeros_nw)
        h = _update(aggp[0], aggp[1], d0, d1, h, root_W, root_b.reshape(1, -1))
        if layer + 1 < _DEPTH:
            xj = _sc_gather(h, src)
            msg = _matvec(xj, K)

    return _fc2(h, fc2_W.reshape(1, _W), fc2_b.reshape(1, 1))


# speedup vs baseline: 5.9180x; 1.0259x over previous
import functools

import jax
import jax.numpy as jnp
import numpy as np
from jax import lax
from jax.experimental import pallas as pl
from jax.experimental.pallas import tpu as pltpu
from jax.experimental.pallas import tpu_sc as plsc

_N = 10000
_E = 160000
_W = 64
_DEPTH = 3

_NC = 2
_NS = 16
_NW = _NC * _NS
_EPW = _E // _NW
_GCH = 128
_NFULL = _EPW // _GCH
_TAIL = _EPW - _NFULL * _GCH
_RPT = _N // _NS
_SCH = 512
_SUP = [(j * _SCH, _SCH) for j in range(9)] + [(9 * _SCH, _EPW - 9 * _SCH)]


def _subchunks(size):
    subs = []
    off = 0
    while off < size:
        n = min(_GCH, size - off)
        subs.append((off, n))
        off += n
    return subs

_PERM = np.empty((4096,), np.int32)
for _i in range(32):
    for _p in range(2):
        for _g in range(64):
            _PERM[128 * _i + 64 * _p + _g] = (2 * _i + _p) * 64 + _g

_BX = np.zeros((64, 4096), np.float32)
for _i in range(32):
    for _p in range(2):
        _BX[2 * _i + _p, 128 * _i + 64 * _p:128 * _i + 64 * _p + 64] = 1.0

_EB = 128

_KF_BLK = 640
_MV_BLK = 3200
_ND_BLK = 1000


def _kform_body(ea, k1w, k1b, k2w, k2b, k3w, k3b, xj, B, outk, outm):
    t = jnp.maximum(jnp.dot(ea[...], k1w[...], preferred_element_type=jnp.float32) + k1b[...], 0.0)
    t = jnp.maximum(jnp.dot(t, k2w[...], preferred_element_type=jnp.float32) + k2b[...], 0.0)
    Kf = jnp.dot(t.astype(jnp.bfloat16), k3w[...],
                 preferred_element_type=jnp.float32) + k3b[...]
    outk[...] = Kf.astype(jnp.bfloat16)
    for eb in range(_KF_BLK // _EB):
        x = xj[pl.ds(eb * _EB, _EB), :]
        acc = jnp.zeros((_EB, 2 * _W), jnp.float32)
        for i in range(_W // 2):
            xi = jnp.dot(x, B[:, 128 * i:128 * (i + 1)],
                         preferred_element_type=jnp.float32)
            acc = acc + xi * Kf[eb * _EB:(eb + 1) * _EB, 128 * i:128 * (i + 1)]
        outm[pl.ds(eb * _EB, _EB), :] = acc[:, :_W] + acc[:, _W:]


def _kform(ea, k1w, k1b, k2w, k2b, k3w, k3b, xj):
    n_blk = _E // _KF_BLK
    full = lambda *shape: pl.BlockSpec(shape, lambda i, s=len(shape): (0,) * s)
    return pl.pallas_call(
        _kform_body,
        grid=(n_blk,),
        in_specs=[
            pl.BlockSpec((_KF_BLK, 16), lambda i: (i, 0)),
            full(16, 256), full(1, 256),
            full(256, 512), full(1, 512),
            full(512, 4096), full(1, 4096),
            pl.BlockSpec((_KF_BLK, _W), lambda i: (i, 0)),
            full(_W, 4096),
        ],
        out_specs=(pl.BlockSpec((_KF_BLK, 4096), lambda i: (i, 0)),
                   pl.BlockSpec((_KF_BLK, _W), lambda i: (i, 0))),
        out_shape=(jax.ShapeDtypeStruct((_E, 4096), jnp.bfloat16),
                   jax.ShapeDtypeStruct((_E, _W), jnp.float32)),
    )(ea, k1w, k1b, k2w, k2b, k3w, k3b, xj, jnp.asarray(_BX))


def _matvec_body(xj, K, B, out):
    for eb in range(_MV_BLK // _EB):
        x = xj[pl.ds(eb * _EB, _EB), :]
        acc = jnp.zeros((_EB, 2 * _W), jnp.float32)
        for i in range(_W // 2):
            xi = jnp.dot(x, B[:, 128 * i:128 * (i + 1)],
                         preferred_element_type=jnp.float32)
            kk = K[pl.ds(eb * _EB, _EB), 128 * i:128 * (i + 1)].astype(jnp.float32)
            acc = acc + xi * kk
        out[pl.ds(eb * _EB, _EB), :] = acc[:, :_W] + acc[:, _W:]


def _matvec(xj, K):
    n_blk = _E // _MV_BLK
    return pl.pallas_call(
        _matvec_body,
        grid=(n_blk,),
        in_specs=[
            pl.BlockSpec((_MV_BLK, _W), lambda i: (i, 0)),
            pl.BlockSpec((_MV_BLK, 4096), lambda i: (i, 0)),
            pl.BlockSpec((_W, 4096), lambda i: (0, 0)),
        ],
        out_specs=pl.BlockSpec((_MV_BLK, _W), lambda i: (i, 0)),
        out_shape=jax.ShapeDtypeStruct((_E, _W), jnp.float32),
    )(xj, K, jnp.asarray(_BX))


def _update_body(a0, a1, d0, d1, h, rootw, rootb, out):
    hr = jnp.dot(h[...], rootw[...], preferred_element_type=jnp.float32) + rootb[...]
    deg = jnp.maximum(d0[...] + d1[...], 1.0)
    out[...] = jnp.maximum((a0[...] + a1[...]) / deg + hr, 0.0)


def _update(a0, a1, d0, d1, h, rootw, rootb):
    n_blk = _N // _ND_BLK
    nd = pl.BlockSpec((_ND_BLK, _W), lambda i: (i, 0))
    return pl.pallas_call(
        _update_body,
        grid=(n_blk,),
        in_specs=[
            nd, nd, nd, nd, nd,
            pl.BlockSpec((_W, _W), lambda i: (0, 0)),
            pl.BlockSpec((1, _W), lambda i: (0, 0)),
        ],
        out_specs=pl.BlockSpec((_ND_BLK, _W), lambda i: (i, 0)),
        out_shape=jax.ShapeDtypeStruct((_N, _W), jnp.float32),
    )(a0, a1, d0, d1, h, rootw, rootb)


def _h0_body(x, w, b, out):
    out[...] = x[...] * w[...] + b[...]


def _h0(x, w, b):
    n_blk = _N // _ND_BLK
    return pl.pallas_call(
        _h0_body,
        grid=(n_blk,),
        in_specs=[
            pl.BlockSpec((_ND_BLK, 1), lambda i: (i, 0)),
            pl.BlockSpec((1, _W), lambda i: (0, 0)),
            pl.BlockSpec((1, _W), lambda i: (0, 0)),
        ],
        out_specs=pl.BlockSpec((_ND_BLK, _W), lambda i: (i, 0)),
        out_shape=jax.ShapeDtypeStruct((_N, _W), jnp.float32),
    )(x, w, b)


def _fc2_body(h, w, b, out):
    out[...] = jnp.sum(h[...] * w[...], axis=1, keepdims=True) + b[...]


def _fc2(h, w, b):
    n_blk = _N // _ND_BLK
    return pl.pallas_call(
        _fc2_body,
        grid=(n_blk,),
        in_specs=[
            pl.BlockSpec((_ND_BLK, _W), lambda i: (i, 0)),
            pl.BlockSpec((1, _W), lambda i: (0, 0)),
            pl.BlockSpec((1, 1), lambda i: (0, 0)),
        ],
        out_specs=pl.BlockSpec((_ND_BLK, 1), lambda i: (i, 0)),
        out_shape=jax.ShapeDtypeStruct((_N, 1), jnp.float32),
    )(h, w, b)


def _gather_body(h_hbm, src_hbm, out_hbm, idx3, idxt, rows2, sem_i, sem_g,
                 sem_s0, sem_s1):
    wid = lax.axis_index("s") * _NC + lax.axis_index("c")
    base = wid * _EPW
    ssem = (sem_s0, sem_s1)
    idx_descs = {}
    store_descs = {}

    def issue_idx(j):
        b = j % 2
        off0, size = _SUP[j]
        ds_ = []
        for k, (soff, n) in enumerate(_subchunks(size)):
            dst = idx3.at[b, k] if n == _GCH else idxt
            ds_.append(pltpu.async_copy(
                src_hbm.at[pl.ds(base + off0 + soff, n)], dst, sem_i))
        idx_descs[j] = ds_

    issue_idx(0)
    for j in range(len(_SUP)):
        b = j % 2
        off0, size = _SUP[j]
        if j - 2 in store_descs:
            store_descs.pop(j - 2).wait()
        for d in idx_descs.pop(j):
            d.wait()
        if j + 1 < len(_SUP):
            issue_idx(j + 1)
        g_descs = []
        for k, (soff, n) in enumerate(_subchunks(size)):
            idx = idx3.at[b, k] if n == _GCH else idxt
            g_descs.append(pltpu.async_copy(
                h_hbm.at[idx], rows2.at[b, pl.ds(soff, n)], sem_g))
        for d in g_descs:
            d.wait()
        store_descs[j] = pltpu.async_copy(
            rows2.at[b, pl.ds(0, size)], out_hbm.at[pl.ds(base + off0, size)],
            ssem[b])
    for j in sorted(store_descs):
        store_descs[j].wait()


def _scatter_body(msg_hbm, dst_hbm, zeros_hbm, out_hbm, idx3, idxt, msg2,
                  agg_sh, sem_i, sem_m, sem_a):
    c = lax.axis_index("c")
    s = lax.axis_index("s")
    wid = s * _NC + c
    base = wid * _EPW
    idx_descs = {}
    msg_descs = {}
    add_descs = {}

    def issue_loads(j):
        b = j % 2
        off0, size = _SUP[j]
        di, dm = [], []
        for k, (soff, n) in enumerate(_subchunks(size)):
            dst = idx3.at[b, k] if n == _GCH else idxt
            di.append(pltpu.async_copy(
                dst_hbm.at[pl.ds(base + off0 + soff, n)], dst, sem_i))
        dm.append(pltpu.async_copy(
            msg_hbm.at[pl.ds(base + off0, size)], msg2.at[b, pl.ds(0, size)],
            sem_m))
        idx_descs[j] = di
        msg_descs[j] = dm

    issue_loads(0)
    pltpu.sync_copy(zeros_hbm.at[pl.ds(s * _RPT, _RPT)],
                    agg_sh.at[pl.ds(s * _RPT, _RPT)])
    plsc.subcore_barrier()
    for j in range(len(_SUP)):
        b = j % 2
        off0, size = _SUP[j]
        if j - 1 in add_descs:
            for d in add_descs.pop(j - 1):
                d.wait()
        for d in idx_descs.pop(j) + msg_descs.pop(j):
            d.wait()
        if j + 1 < len(_SUP):
            issue_loads(j + 1)
        ads = []
        for k, (soff, n) in enumerate(_subchunks(size)):
            idx = idx3.at[b, k] if n == _GCH else idxt
            ads.append(pltpu.async_copy(
                msg2.at[b, pl.ds(soff, n)], agg_sh.at[idx], sem_a, add=True))
        add_descs[j] = ads
    for j in sorted(add_descs):
        for d in add_descs.pop(j):
            d.wait()
    plsc.subcore_barrier()
    pltpu.sync_copy(agg_sh.at[pl.ds(s * _RPT, _RPT)],
                    out_hbm.at[c, pl.ds(s * _RPT, _RPT)])


@functools.cache
def _sc_kernels():
    mesh = plsc.VectorSubcoreMesh(core_axis_name="c", subcore_axis_name="s",
                                  num_cores=_NC, num_subcores=_NS)
    nsub = _EPW // _SCH + 1
    gather = pl.kernel(
        _gather_body,
        out_type=jax.ShapeDtypeStruct((_E, _W), jnp.float32),
        mesh=mesh,
        compiler_params=pltpu.CompilerParams(use_tc_tiling_on_sc=False),
        scratch_types=[
            pltpu.VMEM((2, _SCH // _GCH, _GCH), jnp.int32),
            pltpu.VMEM((_TAIL,), jnp.int32),
            pltpu.VMEM((2, _SCH, _W), jnp.float32),
            pltpu.SemaphoreType.DMA,
            pltpu.SemaphoreType.DMA,
            pltpu.SemaphoreType.DMA,
            pltpu.SemaphoreType.DMA,
        ],
    )
    scatter = pl.kernel(
        _scatter_body,
        out_type=jax.ShapeDtypeStruct((_NC, _N, _W), jnp.float32),
        mesh=mesh,
        compiler_params=pltpu.CompilerParams(use_tc_tiling_on_sc=False),
        scratch_types=[
            pltpu.VMEM((2, _SCH // _GCH, _GCH), jnp.int32),
            pltpu.VMEM((_TAIL,), jnp.int32),
            pltpu.VMEM((2, _SCH, _W), jnp.float32),
            pltpu.VMEM_SHARED((_N, _W), jnp.float32),
            pltpu.SemaphoreType.DMA,
            pltpu.SemaphoreType.DMA,
            pltpu.SemaphoreType.DMA,
        ],
    )
    return gather, scatter


def kernel(x, edge_index, edge_attr, fc1_W, fc1_b, k1_W, k1_b, k2_W, k2_b,
           k3_W, k3_b, root_W, root_b, fc2_W, fc2_b):
    src = edge_index[0]
    dst = edge_index[1]

    h = _h0(x, fc1_W, fc1_b.reshape(1, -1))

    _sc_gather, _sc_scatter = _sc_kernels()
    zeros_nw = jnp.zeros((_N, _W), jnp.float32)
    ones_ew = jnp.ones((_E, _W), jnp.float32)
    degp = _sc_scatter(ones_ew, dst, zeros_nw)
    d0, d1 = degp[0], degp[1]

    xj = _sc_gather(h, src)
    K, msg = _kform(edge_attr, k1_W, k1_b.reshape(1, -1), k2_W,
                    k2_b.reshape(1, -1), k3_W[:, _PERM].astype(jnp.bfloat16),
                    k3_b[_PERM].reshape(1, -1), xj)

    for layer in range(_DEPTH):
        aggp = _sc_scatter(msg, dst, zeros_nw)
        h = _update(aggp[0], aggp[1], d0, d1, h, root_W, root_b.reshape(1, -1))
        if layer + 1 < _DEPTH:
            xj = _sc_gather(h, src)
            msg = _matvec(xj, K)

    return _fc2(h, fc2_W.reshape(1, _W), fc2_b.reshape(1, 1))
